# baseline (speedup 1.0000x reference)
import functools

import jax
import jax.numpy as jnp
from jax import lax
from jax.experimental import pallas as pl
from jax.experimental.pallas import tpu as pltpu
from jax.experimental.pallas import tpu_sc as plsc

D = 2048
KSEL = 256
B = 8
HW = 1024
SIGMA = 0.1

R = 16
C = 128

NW = 32
RPW = (B * KSEL) // NW
LANES = 16


def _select_body(mu_ref, noise_ref, extra_ref, topk_ref, scale_ref, rows_ref):
    z = mu_ref[...] + SIGMA * (noise_ref[...] + 0.25 * extra_ref[...])
    gate = jnp.clip(z + 0.5, 0.0, 1.0)

    bits = lax.bitcast_convert_type(gate, jnp.int32)
    bits = jnp.where(bits < 0, 0, bits)

    def bs_step(i, lo):
        cand = lo | (1 << (30 - i))
        cnt = jnp.sum((bits >= cand).astype(jnp.int32))
        return jnp.where(cnt >= KSEL, cand, lo)

    thresh = lax.fori_loop(0, 31, bs_step, jnp.int32(0))
    maskf = (bits >= thresh).astype(jnp.float32)

    iota_c = lax.broadcasted_iota(jnp.int32, (C, C), 0)
    jota_c = lax.broadcasted_iota(jnp.int32, (C, C), 1)
    upper = (iota_c <= jota_c).astype(jnp.float32)
    rowcs = jnp.dot(maskf, upper, preferred_element_type=jnp.float32)
    rowtot = rowcs[:, C - 1:C]
    iota_r = lax.broadcasted_iota(jnp.int32, (R, R), 0)
    jota_r = lax.broadcasted_iota(jnp.int32, (R, R), 1)
    strict = (jota_r < iota_r).astype(jnp.float32)
    prefix = jnp.dot(strict, rowtot, preferred_element_type=jnp.float32)
    ranks = (rowcs + prefix) * maskf

    jcol = lax.broadcasted_iota(jnp.int32, (KSEL, C), 0).astype(jnp.float32) + 1.0
    cidx = lax.broadcasted_iota(jnp.int32, (KSEL, C), 1).astype(jnp.float32)
    topk_acc = jnp.zeros((KSEL, 1), jnp.float32)
    scale_acc = jnp.zeros((KSEL, 1), jnp.float32)
    for r in range(R):
        rank_row = jnp.broadcast_to(ranks[r:r + 1, :], (KSEL, C))
        gate_row = jnp.broadcast_to(gate[r:r + 1, :], (KSEL, C))
        hit = rank_row == jcol
        topk_acc += jnp.sum(
            jnp.where(hit, cidx + float(r * C), 0.0), axis=1, keepdims=True)
        scale_acc += jnp.sum(
            jnp.where(hit, gate_row, 0.0), axis=1, keepdims=True)

    topk_i = topk_acc.astype(jnp.int32)
    topk_ref[...] = topk_i
    scale_ref[...] = jnp.broadcast_to(scale_acc, (KSEL, LANES))
    bgrid = lax.broadcasted_iota(jnp.int32, (KSEL, B), 1) * D
    rows_ref[...] = topk_i + bgrid


def _select(mu, noise, extra):
    grid = lambda a: a.reshape(R, C)
    return pl.pallas_call(
        _select_body,
        out_shape=(
            jax.ShapeDtypeStruct((KSEL, 1), jnp.int32),
            jax.ShapeDtypeStruct((KSEL, LANES), jnp.float32),
            jax.ShapeDtypeStruct((KSEL, B), jnp.int32),
        ),
    )(grid(mu), grid(noise), grid(extra))


def _sc_gather(x_flat, rows_flat, scale):
    mesh = plsc.VectorSubcoreMesh(core_axis_name="c", subcore_axis_name="s")

    @functools.partial(
        pl.kernel,
        out_type=jax.ShapeDtypeStruct((B * KSEL, 8, 128), jnp.float32),
        mesh=mesh,
        scratch_types=[
            pltpu.VMEM((RPW,), jnp.int32),
            pltpu.VMEM((RPW, LANES), jnp.float32),
            pltpu.VMEM((RPW, 8, 128), jnp.float32),
            pltpu.SemaphoreType.DMA,
        ],
    )
    def k(x_hbm, rows_hbm, scale_hbm, out_hbm, idx_v, scale_v, rows_v, sem):
        wid = lax.axis_index("s") * 2 + lax.axis_index("c")
        base = wid * RPW
        j0 = (wid % (KSEL // RPW)) * RPW
        pltpu.sync_copy(rows_hbm.at[pl.ds(base, RPW)], idx_v)
        pltpu.sync_copy(scale_hbm.at[pl.ds(j0, RPW), :], scale_v)
        pltpu.async_copy(x_hbm.at[idx_v], rows_v, sem).wait()

        def row_body(r, _):
            s = scale_v[r, :]
            def plane_row_body(i, _):
                for l in range(8):
                    sl = pl.ds(l * LANES, LANES)
                    rows_v[r, i, sl] = rows_v[r, i, sl] * s
                return 0
            return lax.fori_loop(0, 8, plane_row_body, 0)

        lax.fori_loop(0, RPW, row_body, 0)
        pltpu.sync_copy(rows_v, out_hbm.at[pl.ds(base, RPW)])

    return k(x_flat, rows_flat, scale)


def kernel(x, mu, noise, extra_noise):
    x_flat = x.reshape(B * D, 8, 128)
    topk, scale, rows = _select(mu.reshape(R, C), noise.reshape(R, C),
                                extra_noise.reshape(R, C))
    rows_flat = rows.T.reshape(B * KSEL)
    out = _sc_gather(x_flat, rows_flat, scale)
    return out.reshape(B, 1, KSEL, 32, 32)


# speedup vs baseline: 7.1486x; 7.1486x over previous
import jax
import jax.numpy as jnp
from jax import lax
from jax.experimental import pallas as pl
from jax.experimental.pallas import tpu as pltpu

D = 2048
KSEL = 256
B = 8
NPIX = B * 32 * 32
SIGMA = 0.1

A = 128
G = 16

BLK = 1024


def _select_body(mu_ref, noise_ref, extra_ref, et_ref):
    z = mu_ref[...] + SIGMA * (noise_ref[...] + 0.25 * extra_ref[...])
    gate = jnp.clip(z + 0.5, 0.0, 1.0)

    bits = lax.bitcast_convert_type(gate, jnp.int32)
    bits = jnp.where(bits < 0, 0, bits)

    def bs_step(i, lo):
        cand = lo | (1 << (30 - i))
        cnt = jnp.sum((bits >= cand).astype(jnp.int32))
        return jnp.where(cnt >= KSEL, cand, lo)

    thresh = lax.fori_loop(0, 31, bs_step, jnp.int32(0))
    maskf = (bits >= thresh).astype(jnp.float32)

    ia = lax.broadcasted_iota(jnp.int32, (A, A), 0)
    ja = lax.broadcasted_iota(jnp.int32, (A, A), 1)
    lower = (ja <= ia).astype(jnp.float32)
    colcs = jnp.dot(lower, maskf, preferred_element_type=jnp.float32)
    coltot = colcs[A - 1:A, :]
    ig = lax.broadcasted_iota(jnp.int32, (G, G), 0)
    jg = lax.broadcasted_iota(jnp.int32, (G, G), 1)
    strict = (ig < jg).astype(jnp.float32)
    prefix = jnp.dot(coltot, strict, preferred_element_type=jnp.float32)
    ranks = (colcs + prefix) * maskf

    jlane = lax.broadcasted_iota(jnp.int32, (A, KSEL), 1).astype(jnp.float32)
    for g in range(G):
        rank_col = jnp.broadcast_to(ranks[:, g:g + 1], (A, KSEL))
        gate_col = jnp.broadcast_to(gate[:, g:g + 1], (A, KSEL))
        hit = rank_col == jlane + 1.0
        et_ref[pl.ds(g * A, A), :] = jnp.where(
            hit, gate_col, 0.0).astype(jnp.bfloat16)


def _select(mu, noise, extra):
    grid = lambda a: a.reshape(G, A).T
    return pl.pallas_call(
        _select_body,
        out_shape=jax.ShapeDtypeStruct((D, KSEL), jnp.bfloat16),
    )(grid(mu), grid(noise), grid(extra))


def _matmul_body(x_ref, et_ref, out_ref):
    out_ref[...] = lax.dot_general(
        x_ref[...].astype(jnp.bfloat16), et_ref[...],
        (((1,), (0,)), ((), ())), preferred_element_type=jnp.float32)


def _matmul(x2, et):
    return pl.pallas_call(
        _matmul_body,
        grid=(NPIX // BLK,),
        in_specs=[
            pl.BlockSpec((BLK, D), lambda i: (i, 0)),
            pl.BlockSpec((D, KSEL), lambda i: (0, 0)),
        ],
        out_specs=pl.BlockSpec((BLK, KSEL), lambda i: (i, 0)),
        out_shape=jax.ShapeDtypeStruct((NPIX, KSEL), jnp.float32),
    )(x2, et)


def kernel(x, mu, noise, extra_noise):
    x2 = x.reshape(B, D, 32, 32).transpose(0, 2, 3, 1).reshape(NPIX, D)
    et = _select(mu, noise, extra_noise)
    out2 = _matmul(x2, et)
    return out2.reshape(B, 32, 32, KSEL).transpose(0, 3, 1, 2)[:, None]
